# baseline (speedup 1.0000x reference)
import functools
import math

import jax
import jax.numpy as jnp
from jax import lax
from jax.experimental import pallas as pl
from jax.experimental.pallas import tpu as pltpu
from jax.experimental.pallas import tpu_sc as plsc

_RADIAL_CUTOFF = 0.5
_MIN_DISTANCE = 0.0
_N_RBF = 20
_EMB_DIM = 64
_NUM_ATOMS = 100000
_NUM_PAIRS = 3200000


_BLK = 8000


def _rbf_body(d_ref, fij_ref, fcut_ref):
    d = d_ref[...]
    centers = lax.broadcasted_iota(jnp.int32, (1, _N_RBF), 1).astype(jnp.float32) * (
        (_RADIAL_CUTOFF - _MIN_DISTANCE) / (_N_RBF - 1)
    )
    inv_scale = _N_RBF / (_RADIAL_CUTOFF - _MIN_DISTANCE)
    x = (d - centers) * inv_scale
    fij_ref[...] = jnp.exp(-0.5 * (x * x))
    fc = 0.5 * (jnp.cos(d * (math.pi / _RADIAL_CUTOFF)) + 1.0)
    fcut_ref[...] = fc * (d < _RADIAL_CUTOFF).astype(jnp.float32)


def _rbf_cutoff(d_ij):
    grid = _NUM_PAIRS // _BLK
    return pl.pallas_call(
        _rbf_body,
        grid=(grid,),
        in_specs=[pl.BlockSpec((_BLK, 1), lambda i: (i, 0))],
        out_specs=[
            pl.BlockSpec((_BLK, _N_RBF), lambda i: (i, 0)),
            pl.BlockSpec((_BLK, 1), lambda i: (i, 0)),
        ],
        out_shape=[
            jax.ShapeDtypeStruct((_NUM_PAIRS, _N_RBF), jnp.float32),
            jax.ShapeDtypeStruct((_NUM_PAIRS, 1), jnp.float32),
        ],
    )(d_ij)


_TILE = 128
_NTILES = -(-_NUM_ATOMS // _TILE)
_LAST_BASE = _NUM_ATOMS - _TILE
_NW = 32
_STEPS = -(-_NTILES // _NW)


def _gather_body(an_hbm, emb_hbm, out_hbm, idx_v, row_v, sem):
    wid = lax.axis_index("s") * 2 + lax.axis_index("c")

    def step(i, carry):
        t = wid + i * _NW

        @pl.when(t < _NTILES)
        def _():
            base = jnp.minimum(t * _TILE, _LAST_BASE)
            pltpu.sync_copy(an_hbm.at[pl.ds(base, _TILE)], idx_v)
            pltpu.async_copy(emb_hbm.at[idx_v], row_v, sem).wait()
            pltpu.sync_copy(row_v, out_hbm.at[pl.ds(base, _TILE)])

        return carry

    lax.fori_loop(0, _STEPS, step, 0)


@functools.cache
def _embedding_gather_kernel():
    return pl.kernel(
        _gather_body,
        out_type=jax.ShapeDtypeStruct((_NUM_ATOMS, _EMB_DIM), jnp.float32),
        mesh=plsc.VectorSubcoreMesh(core_axis_name="c", subcore_axis_name="s"),
        scratch_types=[
            pltpu.VMEM((_TILE,), jnp.int32),
            pltpu.VMEM((_TILE, _EMB_DIM), jnp.float32),
            pltpu.SemaphoreType.DMA,
        ],
        compiler_params=pltpu.CompilerParams(use_tc_tiling_on_sc=False),
    )


def kernel(d_ij, atomic_numbers, embedding_weight):
    f_ij, f_cutoff = _rbf_cutoff(d_ij)
    atomic_embedding = _embedding_gather_kernel()(atomic_numbers, embedding_weight)
    return (f_ij, f_cutoff, atomic_embedding)
